# baseline (device time: 29944 ns/iter reference)
import jax
import jax.numpy as jnp
from jax import lax
from jax.experimental import pallas as pl
from jax.experimental.pallas import tpu as pltpu

N_DEV = 32
M = 512
N = 512
CHUNK = M // N_DEV
HALF = N // 2


def kernel(A, B):
    def body(a_ref, b_ref, out_ref, c3_ref, rs_comm_ref, ag_ref,
             rs_ssems0, rs_rsems0, ag_ssems0, ag_rsems0,
             rs_ssems1, rs_rsems1, ag_ssems1, ag_rsems1):
        my = lax.axis_index("i")

        barrier_sem = pltpu.get_barrier_semaphore()
        pl.semaphore_signal(
            barrier_sem, inc=1, device_id=(my,),
            device_id_type=pl.DeviceIdType.MESH,
        )
        pl.semaphore_wait(barrier_sem, 1)

        a = a_ref[...].astype(jnp.bfloat16)
        b = b_ref[...].astype(jnp.bfloat16)
        partial = jnp.dot(a, b, preferred_element_type=jnp.float32)
        c3_ref[...] = partial.astype(jnp.bfloat16).reshape(N_DEV, CHUNK, N)

        def rs_send(h, ssems, rsems):
            sends = []
            for d in range(1, N_DEV):
                tgt = lax.rem(my + d, N_DEV)
                rdma = pltpu.make_async_remote_copy(
                    src_ref=c3_ref.at[tgt, :, pl.ds(h * HALF, HALF)],
                    dst_ref=rs_comm_ref.at[my, :, pl.ds(h * HALF, HALF)],
                    send_sem=ssems.at[d - 1],
                    recv_sem=rsems.at[my],
                    device_id=(tgt,),
                    device_id_type=pl.DeviceIdType.MESH,
                )
                rdma.start()
                sends.append(rdma)
            return sends

        def rs_wait_accum(h, rsems):
            acc = c3_ref[pl.ds(my, 1), :, pl.ds(h * HALF, HALF)][0]
            acc = acc.astype(jnp.float32)
            for d in range(1, N_DEV):
                src = lax.rem(my - d + N_DEV, N_DEV)
                recv = pltpu.make_async_remote_copy(
                    src_ref=c3_ref.at[0, :, pl.ds(h * HALF, HALF)],
                    dst_ref=rs_comm_ref.at[src, :, pl.ds(h * HALF, HALF)],
                    send_sem=rsems.at[src],
                    recv_sem=rsems.at[src],
                    device_id=(src,),
                    device_id_type=pl.DeviceIdType.MESH,
                )
                recv.wait_recv()
                acc = acc + rs_comm_ref[
                    pl.ds(src, 1), :, pl.ds(h * HALF, HALF)
                ][0].astype(jnp.float32)
            return acc

        def ag_send(h, ssems, rsems):
            sends = []
            for d in range(1, N_DEV):
                tgt = lax.rem(my + d, N_DEV)
                rdma = pltpu.make_async_remote_copy(
                    src_ref=ag_ref.at[my, :, pl.ds(h * HALF, HALF)],
                    dst_ref=ag_ref.at[my, :, pl.ds(h * HALF, HALF)],
                    send_sem=ssems.at[d - 1],
                    recv_sem=rsems.at[my],
                    device_id=(tgt,),
                    device_id_type=pl.DeviceIdType.MESH,
                )
                rdma.start()
                sends.append(rdma)
            return sends

        rs_sends0 = rs_send(0, rs_ssems0, rs_rsems0)
        acc0 = rs_wait_accum(0, rs_rsems0)
        ag_ref[pl.ds(my, 1), :, pl.ds(0, HALF)] = (
            acc0.astype(jnp.bfloat16)[None]
        )
        ag_sends0 = ag_send(0, ag_ssems0, ag_rsems0)
        rs_sends1 = rs_send(1, rs_ssems1, rs_rsems1)

        acc1 = rs_wait_accum(1, rs_rsems1)
        ag_ref[pl.ds(my, 1), :, pl.ds(HALF, HALF)] = (
            acc1.astype(jnp.bfloat16)[None]
        )
        ag_sends1 = ag_send(1, ag_ssems1, ag_rsems1)

        out_ref[pl.ds(my * CHUNK, CHUNK), pl.ds(0, HALF)] = acc0
        out_ref[pl.ds(my * CHUNK, CHUNK), pl.ds(HALF, HALF)] = acc1

        for rdma in rs_sends0 + rs_sends1:
            rdma.wait_send()

        for d in range(1, N_DEV):
            src = lax.rem(my - d + N_DEV, N_DEV)
            for h, rsems in ((0, ag_rsems0), (1, ag_rsems1)):
                recv = pltpu.make_async_remote_copy(
                    src_ref=c3_ref.at[0, :, pl.ds(h * HALF, HALF)],
                    dst_ref=ag_ref.at[src, :, pl.ds(h * HALF, HALF)],
                    send_sem=rsems.at[src],
                    recv_sem=rsems.at[src],
                    device_id=(src,),
                    device_id_type=pl.DeviceIdType.MESH,
                )
                recv.wait_recv()
            out_ref[pl.ds(src * CHUNK, CHUNK), :] = (
                ag_ref[pl.ds(src, 1)][0].astype(jnp.float32)
            )

        for rdma in ag_sends0 + ag_sends1:
            rdma.wait_send()

    return pl.pallas_call(
        body,
        out_shape=jax.ShapeDtypeStruct((M, N), jnp.float32),
        in_specs=[
            pl.BlockSpec(memory_space=pltpu.VMEM),
            pl.BlockSpec(memory_space=pltpu.VMEM),
        ],
        out_specs=pl.BlockSpec(memory_space=pltpu.VMEM),
        compiler_params=pltpu.CompilerParams(collective_id=0),
        scratch_shapes=[
            pltpu.VMEM((N_DEV, CHUNK, N), jnp.bfloat16),
            pltpu.VMEM((N_DEV, CHUNK, N), jnp.bfloat16),
            pltpu.VMEM((N_DEV, CHUNK, N), jnp.bfloat16),
            pltpu.SemaphoreType.DMA((N_DEV,)),
            pltpu.SemaphoreType.DMA((N_DEV,)),
            pltpu.SemaphoreType.DMA((N_DEV,)),
            pltpu.SemaphoreType.DMA((N_DEV,)),
            pltpu.SemaphoreType.DMA((N_DEV,)),
            pltpu.SemaphoreType.DMA((N_DEV,)),
            pltpu.SemaphoreType.DMA((N_DEV,)),
            pltpu.SemaphoreType.DMA((N_DEV,)),
        ],
    )(A, B)


# device time: 24774 ns/iter; 1.2087x vs baseline; 1.2087x over previous
import jax
import jax.numpy as jnp
from jax import lax
from jax.experimental import pallas as pl
from jax.experimental.pallas import tpu as pltpu

N_DEV = 32
M = 512
N = 512
CHUNK = M // N_DEV


def kernel(A, B):
    def body(a_ref, b_ref, out_ref, c3_ref, rs_comm_ref, ag_comm_ref,
             agbuf_ref, rs_send_sems, rs_recv_sems, ag_send_sems,
             ag_recv_sems):
        my = lax.axis_index("i")

        barrier_sem = pltpu.get_barrier_semaphore()
        pl.semaphore_signal(
            barrier_sem, inc=1, device_id=(my,),
            device_id_type=pl.DeviceIdType.MESH,
        )
        pl.semaphore_wait(barrier_sem, 1)

        a = a_ref[...].astype(jnp.bfloat16)
        b = b_ref[...].astype(jnp.bfloat16)
        partial = jnp.dot(a, b, preferred_element_type=jnp.float32)
        c3_ref[...] = partial.astype(jnp.bfloat16).reshape(N_DEV, CHUNK, N)

        rs_sends = []
        for d in range(1, N_DEV):
            tgt = lax.rem(my + d, N_DEV)
            rdma = pltpu.make_async_remote_copy(
                src_ref=c3_ref.at[tgt],
                dst_ref=rs_comm_ref.at[d],
                send_sem=rs_send_sems.at[d],
                recv_sem=rs_recv_sems.at[d],
                device_id=(tgt,),
                device_id_type=pl.DeviceIdType.MESH,
            )
            rdma.start()
            rs_sends.append(rdma)

        acc = c3_ref[pl.ds(my, 1)][0].astype(jnp.float32)
        for d in range(1, N_DEV):
            recv = pltpu.make_async_remote_copy(
                src_ref=c3_ref.at[0],
                dst_ref=rs_comm_ref.at[d],
                send_sem=rs_send_sems.at[0],
                recv_sem=rs_recv_sems.at[d],
                device_id=(my,),
                device_id_type=pl.DeviceIdType.MESH,
            )
            recv.wait_recv()
            acc = acc + rs_comm_ref[d].astype(jnp.float32)

        agbuf_ref[...] = acc.astype(jnp.bfloat16)

        ag_sends = []
        for d in range(1, N_DEV):
            tgt = lax.rem(my + d, N_DEV)
            rdma = pltpu.make_async_remote_copy(
                src_ref=agbuf_ref,
                dst_ref=ag_comm_ref.at[d],
                send_sem=ag_send_sems.at[d],
                recv_sem=ag_recv_sems.at[d],
                device_id=(tgt,),
                device_id_type=pl.DeviceIdType.MESH,
            )
            rdma.start()
            ag_sends.append(rdma)

        for rdma in rs_sends:
            rdma.wait_send()

        out_ref[pl.ds(my * CHUNK, CHUNK), :] = acc

        for d in range(1, N_DEV):
            src = lax.rem(my - d + N_DEV, N_DEV)
            recv = pltpu.make_async_remote_copy(
                src_ref=agbuf_ref,
                dst_ref=ag_comm_ref.at[d],
                send_sem=ag_send_sems.at[0],
                recv_sem=ag_recv_sems.at[d],
                device_id=(my,),
                device_id_type=pl.DeviceIdType.MESH,
            )
            recv.wait_recv()
            out_ref[pl.ds(src * CHUNK, CHUNK), :] = (
                ag_comm_ref[d].astype(jnp.float32)
            )

        for rdma in ag_sends:
            rdma.wait_send()

    return pl.pallas_call(
        body,
        out_shape=jax.ShapeDtypeStruct((M, N), jnp.float32),
        in_specs=[
            pl.BlockSpec(memory_space=pltpu.VMEM),
            pl.BlockSpec(memory_space=pltpu.VMEM),
        ],
        out_specs=pl.BlockSpec(memory_space=pltpu.VMEM),
        compiler_params=pltpu.CompilerParams(collective_id=0),
        scratch_shapes=[
            pltpu.VMEM((N_DEV, CHUNK, N), jnp.bfloat16),
            pltpu.VMEM((N_DEV, CHUNK, N), jnp.bfloat16),
            pltpu.VMEM((N_DEV, CHUNK, N), jnp.bfloat16),
            pltpu.VMEM((CHUNK, N), jnp.bfloat16),
            pltpu.SemaphoreType.DMA((N_DEV,)),
            pltpu.SemaphoreType.DMA((N_DEV,)),
            pltpu.SemaphoreType.DMA((N_DEV,)),
            pltpu.SemaphoreType.DMA((N_DEV,)),
        ],
    )(A, B)


# device time: 23474 ns/iter; 1.2756x vs baseline; 1.0554x over previous
import jax
import jax.numpy as jnp
from jax import lax
from jax.experimental import pallas as pl
from jax.experimental.pallas import tpu as pltpu

N_DEV = 32
M = 512
N = 512
CHUNK = M // N_DEV
N_BLK = 4
BLK_ROWS = M // N_BLK
BLK_CHUNKS = N_DEV // N_BLK
GRP = N_DEV // N_BLK


def kernel(A, B):
    def body(a_ref, b_ref, out_ref, c3_ref, rs_comm_ref, ag_comm_ref,
             agbuf_ref, rs_send_sems, rs_recv_sems, ag_send_sems,
             ag_recv_sems):
        my = lax.axis_index("i")
        my_grp = lax.div(my, GRP)

        barrier_sem = pltpu.get_barrier_semaphore()
        pl.semaphore_signal(
            barrier_sem, inc=1, device_id=(my,),
            device_id_type=pl.DeviceIdType.MESH,
        )
        pl.semaphore_wait(barrier_sem, 1)

        b = b_ref[...].astype(jnp.bfloat16)

        for w in range(N_BLK):
            rb = lax.rem(my_grp + 1 + w, N_BLK)
            a_blk = a_ref[pl.ds(rb * BLK_ROWS, BLK_ROWS), :].astype(
                jnp.bfloat16
            )
            blk = jnp.dot(a_blk, b, preferred_element_type=jnp.float32)
            c3_ref[pl.ds(rb * BLK_CHUNKS, BLK_CHUNKS)] = (
                blk.astype(jnp.bfloat16).reshape(BLK_CHUNKS, CHUNK, N)
            )
            for k in range(BLK_CHUNKS):
                c = rb * BLK_CHUNKS + k

                @pl.when(c != my)
                def _(c=c, w=w, k=k):
                    rdma = pltpu.make_async_remote_copy(
                        src_ref=c3_ref.at[c],
                        dst_ref=rs_comm_ref.at[my],
                        send_sem=rs_send_sems.at[w * BLK_CHUNKS + k],
                        recv_sem=rs_recv_sems.at[my],
                        device_id=(c,),
                        device_id_type=pl.DeviceIdType.MESH,
                    )
                    rdma.start()

                @pl.when(c == my)
                def _():
                    pltpu.make_async_copy(
                        c3_ref.at[my], rs_comm_ref.at[my],
                        rs_recv_sems.at[my],
                    ).start()

        acc = jnp.zeros((CHUNK, N), jnp.float32)
        for w in range(N_BLK):
            src_grp = lax.rem(my_grp + (N_BLK - 1) - w, N_BLK)
            for j in range(GRP):
                src = src_grp * GRP + j
                pltpu.make_async_copy(
                    c3_ref.at[0], rs_comm_ref.at[src], rs_recv_sems.at[src]
                ).wait()
                acc = acc + rs_comm_ref[pl.ds(src, 1)][0].astype(jnp.float32)

        agbuf_ref[...] = acc.astype(jnp.bfloat16)

        ag_sends = []
        for d in range(1, N_DEV):
            tgt = lax.rem(my + d, N_DEV)
            rdma = pltpu.make_async_remote_copy(
                src_ref=agbuf_ref,
                dst_ref=ag_comm_ref.at[d],
                send_sem=ag_send_sems.at[d],
                recv_sem=ag_recv_sems.at[d],
                device_id=(tgt,),
                device_id_type=pl.DeviceIdType.MESH,
            )
            rdma.start()
            ag_sends.append(rdma)

        for w in range(N_BLK):
            rb = lax.rem(my_grp + 1 + w, N_BLK)
            for k in range(BLK_CHUNKS):
                c = rb * BLK_CHUNKS + k

                @pl.when(c != my)
                def _(c=c, w=w, k=k):
                    pltpu.make_async_remote_copy(
                        src_ref=c3_ref.at[c],
                        dst_ref=rs_comm_ref.at[my],
                        send_sem=rs_send_sems.at[w * BLK_CHUNKS + k],
                        recv_sem=rs_recv_sems.at[my],
                        device_id=(c,),
                        device_id_type=pl.DeviceIdType.MESH,
                    ).wait_send()

        out_ref[pl.ds(my * CHUNK, CHUNK), :] = acc

        for d in range(1, N_DEV):
            src = lax.rem(my - d + N_DEV, N_DEV)
            recv = pltpu.make_async_remote_copy(
                src_ref=agbuf_ref,
                dst_ref=ag_comm_ref.at[d],
                send_sem=ag_send_sems.at[0],
                recv_sem=ag_recv_sems.at[d],
                device_id=(my,),
                device_id_type=pl.DeviceIdType.MESH,
            )
            recv.wait_recv()
            out_ref[pl.ds(src * CHUNK, CHUNK), :] = (
                ag_comm_ref[d].astype(jnp.float32)
            )

        for rdma in ag_sends:
            rdma.wait_send()

    return pl.pallas_call(
        body,
        out_shape=jax.ShapeDtypeStruct((M, N), jnp.float32),
        in_specs=[
            pl.BlockSpec(memory_space=pltpu.VMEM),
            pl.BlockSpec(memory_space=pltpu.VMEM),
        ],
        out_specs=pl.BlockSpec(memory_space=pltpu.VMEM),
        compiler_params=pltpu.CompilerParams(collective_id=0),
        scratch_shapes=[
            pltpu.VMEM((N_DEV, CHUNK, N), jnp.bfloat16),
            pltpu.VMEM((N_DEV, CHUNK, N), jnp.bfloat16),
            pltpu.VMEM((N_DEV, CHUNK, N), jnp.bfloat16),
            pltpu.VMEM((CHUNK, N), jnp.bfloat16),
            pltpu.SemaphoreType.DMA((N_DEV,)),
            pltpu.SemaphoreType.DMA((N_DEV,)),
            pltpu.SemaphoreType.DMA((N_DEV,)),
            pltpu.SemaphoreType.DMA((N_DEV,)),
        ],
    )(A, B)
